# baseline (speedup 1.0000x reference)
import functools

import jax
import jax.numpy as jnp
from jax import lax
from jax.experimental import pallas as pl
from jax.experimental.pallas import tpu as pltpu
from jax.experimental.pallas import tpu_sc as plsc

NUM_EMB = 1000000
H = 64
BATCH = 4096
HIST = 200

NC = 2
NS = 16
NW = NC * NS

N = BATCH * HIST
M = N // NW
C = 128
SUB = C // 16
K = M // C
NBUF = 4
LAG = 2
T = K // NBUF


def _make_gather():
    mesh = plsc.VectorSubcoreMesh(core_axis_name="c", subcore_axis_name="s")

    @functools.partial(
        pl.kernel,
        mesh=mesh,
        out_type=jax.ShapeDtypeStruct((N // 2, 2 * H), jnp.float32),
        scratch_types=[
            pltpu.VMEM((K, C), jnp.int32),
            pltpu.VMEM((NBUF, C, 2 * H), jnp.float32),
            pltpu.VMEM((NBUF, C // 2, 2 * H), jnp.float32),
            pltpu.SemaphoreType.DMA((NBUF,)),
            pltpu.SemaphoreType.DMA((NBUF,)),
        ],
    )
    def k(table_hbm, idx_hbm, out_hbm, idx_v, rows_v, pack_v, gsem, osem):
        wid = lax.axis_index("s") * NC + lax.axis_index("c")
        base2 = wid * (M // 2)
        pltpu.sync_copy(idx_hbm.at[pl.ds(wid * K, K)], idx_v)

        def gather_descs(j, slot):
            descs = []
            for u in range(SUB):
                vec = idx_v[j, pl.ds(u * 16, 16)] >> 1
                descs.append(pltpu.make_async_copy(
                    table_hbm.at[vec],
                    rows_v.at[slot, pl.ds(u * 16, 16)],
                    gsem.at[slot],
                ))
            return descs

        def store_desc(j, slot):
            return pltpu.make_async_copy(
                pack_v.at[slot],
                out_hbm.at[pl.ds(base2 + j * (C // 2), C // 2)],
                osem.at[slot],
            )

        def compact(j, slot):
            def body(g, carry):
                par16 = (idx_v[j, pl.ds(g * 16, 16)] & 1) * H
                for l in range(16):
                    off = par16[l]
                    for q in range(H // 16):
                        pack_v[slot, g * 8 + l // 2,
                               pl.ds((l % 2) * H + q * 16, 16)] = (
                            rows_v[slot, g * 16 + l, pl.ds(off + q * 16, 16)])
                return carry

            lax.fori_loop(0, C // 16, body, 0)

        def round_body(t, carry):
            for b in range(NBUF):
                j = t * NBUF + b

                @pl.when(j >= NBUF)
                def _():
                    store_desc(j - NBUF, b).wait()

                for d in gather_descs(j, b):
                    d.start()

                j2 = j - LAG
                b2 = (b + NBUF - LAG) % NBUF

                @pl.when(j2 >= 0)
                def _():
                    for d in gather_descs(j2, b2):
                        d.wait()
                    compact(j2, b2)
                    store_desc(j2, b2).start()

            return carry

        lax.fori_loop(0, T, round_body, 0)

        for b in range(NBUF - LAG, NBUF):
            j2 = K - NBUF + b
            for d in gather_descs(j2, b):
                d.wait()
            compact(j2, b)
            store_desc(j2, b).start()
        for b in range(NBUF):
            store_desc(K - NBUF + b, b).wait()

    return k


_gather = _make_gather()


@jax.jit
def kernel(input_ids, table):
    ids2 = input_ids.reshape(N // C, C).astype(jnp.int32)
    table2 = table.reshape(NUM_EMB // 2, 2 * H)
    out = _gather(table2, ids2)
    return out.reshape(BATCH, HIST, H)
